# Initial kernel scaffold; baseline (speedup 1.0000x reference)
import jax
import jax.numpy as jnp
from jax.experimental import pallas as pl


def kernel(x, edge_index, W1, b1, W2, b2):
    raise NotImplementedError("write your pallas kernel here")


# speedup vs baseline: 18.6781x; 18.6781x over previous
import functools

import jax
import jax.numpy as jnp
from jax import lax
from jax.experimental import pallas as pl
from jax.experimental.pallas import tpu as pltpu
from jax.experimental.pallas import tpu_sc as plsc

N = 10000
D = 128
E = 320000
NC = 2
NS = 16
NW = NC * NS
EPW = E // NW
CH = 80
NCHUNK = EPW // CH
NWR = 10
RPT = N // NWR
ZR = 40

_MESH = plsc.VectorSubcoreMesh(core_axis_name="c", subcore_axis_name="s")


def _zero_rows(z_ref, nrows, ncol16):
    zv = jnp.zeros((16,), jnp.float32)

    def body(i, _):
        for k in range(ncol16):
            z_ref[i, pl.ds(k * 16, 16)] = zv
        return 0

    lax.fori_loop(0, nrows, body, 0)


@functools.partial(
    pl.kernel,
    out_type=jax.ShapeDtypeStruct((NC, N, 16), jnp.float32),
    mesh=_MESH,
    scratch_types=[
        pltpu.VMEM((NCHUNK, CH), jnp.int32),
        pltpu.VMEM((CH, 16), jnp.float32),
        pltpu.VMEM((ZR, 16), jnp.float32),
        pltpu.VMEM_SHARED((N, 16), jnp.float32),
    ],
)
def _deg_kernel(dst_hbm, out_hbm, dst_v, ones_v, z_v, deg_sh):
    c = lax.axis_index("c")
    s = lax.axis_index("s")

    ones = jnp.ones((16,), jnp.float32)

    def fill_ones(i, _):
        ones_v[i, pl.ds(0, 16)] = ones
        return 0

    lax.fori_loop(0, CH, fill_ones, 0)
    _zero_rows(z_v, ZR, 1)

    @pl.when(s < NWR)
    def _():
        for k in range(RPT // ZR):
            pltpu.sync_copy(z_v, deg_sh.at[pl.ds(s * RPT + k * ZR, ZR)])

    plsc.subcore_barrier()

    pltpu.sync_copy(dst_hbm.at[c, s], dst_v)

    def body(j, _):
        pltpu.sync_copy(ones_v, deg_sh.at[dst_v.at[j]], add=True)
        return 0

    lax.fori_loop(0, NCHUNK, body, 0)
    plsc.subcore_barrier()

    @pl.when(s < NWR)
    def _():
        pltpu.sync_copy(deg_sh.at[pl.ds(s * RPT, RPT)],
                        out_hbm.at[c, pl.ds(s * RPT, RPT)])


@functools.partial(
    pl.kernel,
    out_type=jax.ShapeDtypeStruct((NC, N, D), jnp.float32),
    mesh=_MESH,
    scratch_types=[
        pltpu.VMEM((NCHUNK, CH), jnp.int32),
        pltpu.VMEM((NCHUNK, CH), jnp.int32),
        pltpu.VMEM((CH, D), jnp.float32),
        pltpu.VMEM((ZR, D), jnp.float32),
        pltpu.VMEM_SHARED((N, D), jnp.float32),
        pltpu.SemaphoreType.DMA,
    ],
)
def _segsum_kernel(y_hbm, src_hbm, dst_hbm, out_hbm,
                   src_v, dst_v, rows_v, z_v, acc_sh, sem):
    c = lax.axis_index("c")
    s = lax.axis_index("s")

    _zero_rows(z_v, ZR, D // 16)

    @pl.when(s < NWR)
    def _():
        for k in range(RPT // ZR):
            pltpu.sync_copy(z_v, acc_sh.at[pl.ds(s * RPT + k * ZR, ZR)])

    plsc.subcore_barrier()

    pltpu.sync_copy(src_hbm.at[c, s], src_v)
    pltpu.sync_copy(dst_hbm.at[c, s], dst_v)

    def body(j, _):
        pltpu.async_copy(y_hbm.at[src_v.at[j]], rows_v, sem).wait()
        pltpu.sync_copy(rows_v, acc_sh.at[dst_v.at[j]], add=True)
        return 0

    lax.fori_loop(0, NCHUNK, body, 0)
    plsc.subcore_barrier()

    @pl.when(s < NWR)
    def _():
        pltpu.sync_copy(acc_sh.at[pl.ds(s * RPT, RPT)],
                        out_hbm.at[c, pl.ds(s * RPT, RPT)])


RB = 1000


def _dis(d0_ref, d1_ref):
    deg = d0_ref[...][:, 0:1] + d1_ref[...][:, 0:1] + 1.0
    return lax.rsqrt(deg)


def _mm_scale_body(x_ref, w_ref, d0_ref, d1_ref, y_ref):
    dis = _dis(d0_ref, d1_ref)
    y_ref[...] = jnp.dot(x_ref[...], w_ref[...],
                         preferred_element_type=jnp.float32) * dis


def _mid_body(a0_ref, a1_ref, y1_ref, d0_ref, d1_ref, b_ref, w_ref, y2_ref):
    dis = _dis(d0_ref, d1_ref)
    h = dis * (a0_ref[...] + a1_ref[...] + y1_ref[...]) + b_ref[...]
    h = jnp.maximum(h, 0.0)
    y2_ref[...] = jnp.dot(h, w_ref[...],
                          preferred_element_type=jnp.float32) * dis


def _final_body(a0_ref, a1_ref, y2_ref, d0_ref, d1_ref, b_ref, out_ref):
    dis = _dis(d0_ref, d1_ref)
    out_ref[...] = dis * (a0_ref[...] + a1_ref[...] + y2_ref[...]) + b_ref[...]


def _row_spec():
    return pl.BlockSpec((RB, D), lambda i: (i, 0))


def _deg_spec():
    return pl.BlockSpec((RB, 16), lambda i: (i, 0))


def _full_spec(shape):
    return pl.BlockSpec(shape, lambda i: tuple(0 for _ in shape))


def _mm_scale(x, w, d0, d1):
    return pl.pallas_call(
        _mm_scale_body,
        grid=(N // RB,),
        in_specs=[_row_spec(), _full_spec((D, D)), _deg_spec(), _deg_spec()],
        out_specs=_row_spec(),
        out_shape=jax.ShapeDtypeStruct((N, D), jnp.float32),
    )(x, w, d0, d1)


def _mid(a0, a1, y1, d0, d1, b, w):
    return pl.pallas_call(
        _mid_body,
        grid=(N // RB,),
        in_specs=[_row_spec(), _row_spec(), _row_spec(), _deg_spec(),
                  _deg_spec(), _full_spec((1, D)), _full_spec((D, D))],
        out_specs=_row_spec(),
        out_shape=jax.ShapeDtypeStruct((N, D), jnp.float32),
    )(a0, a1, y1, d0, d1, b, w)


def _final(a0, a1, y2, d0, d1, b):
    return pl.pallas_call(
        _final_body,
        grid=(N // RB,),
        in_specs=[_row_spec(), _row_spec(), _row_spec(), _deg_spec(),
                  _deg_spec(), _full_spec((1, D))],
        out_specs=_row_spec(),
        out_shape=jax.ShapeDtypeStruct((N, D), jnp.float32),
    )(a0, a1, y2, d0, d1, b)


def kernel(x, edge_index, W1, b1, W2, b2):
    src = edge_index[0].astype(jnp.int32).reshape(NC, NS, NCHUNK, CH)
    dst = edge_index[1].astype(jnp.int32).reshape(NC, NS, NCHUNK, CH)
    b1r = b1.reshape(1, D)
    b2r = b2.reshape(1, D)

    degp = _deg_kernel(dst)
    d0, d1 = degp[0], degp[1]

    y1 = _mm_scale(x, W1, d0, d1)
    acc1 = _segsum_kernel(y1, src, dst)
    y2 = _mid(acc1[0], acc1[1], y1, d0, d1, b1r, W2)
    acc2 = _segsum_kernel(y2, src, dst)
    return _final(acc2[0], acc2[1], y2, d0, d1, b2r)


# speedup vs baseline: 26.5772x; 1.4229x over previous
import functools

import jax
import jax.numpy as jnp
from jax import lax
from jax.experimental import pallas as pl
from jax.experimental.pallas import tpu as pltpu
from jax.experimental.pallas import tpu_sc as plsc

N = 10000
D = 128
E = 320000
NC = 2
NS = 16
NW = NC * NS
EPW = E // NW
CH = 80
NCHUNK = EPW // CH
NWR = 10
RPT = N // NWR
ZR = 8

_MESH = plsc.VectorSubcoreMesh(core_axis_name="c", subcore_axis_name="s")


def _zero_rows(z_ref, nrows, ncol16):
    zv = jnp.zeros((16,), jnp.float32)

    def body(i, _):
        for k in range(ncol16):
            z_ref[i, pl.ds(k * 16, 16)] = zv
        return 0

    lax.fori_loop(0, nrows, body, 0)


@functools.partial(
    pl.kernel,
    out_type=jax.ShapeDtypeStruct((NC, N, 16), jnp.float32),
    mesh=_MESH,
    scratch_types=[
        pltpu.VMEM((NCHUNK, CH), jnp.int32),
        pltpu.VMEM((CH, 16), jnp.float32),
        pltpu.VMEM((ZR, 16), jnp.float32),
        pltpu.VMEM_SHARED((N, 16), jnp.float32),
    ],
)
def _deg_kernel(dst_hbm, out_hbm, dst_v, ones_v, z_v, deg_sh):
    c = lax.axis_index("c")
    s = lax.axis_index("s")

    ones = jnp.ones((16,), jnp.float32)

    def fill_ones(i, _):
        ones_v[i, pl.ds(0, 16)] = ones
        return 0

    lax.fori_loop(0, CH, fill_ones, 0)
    _zero_rows(z_v, ZR, 1)

    @pl.when(s < NWR)
    def _():
        for k in range(RPT // ZR):
            pltpu.sync_copy(z_v, deg_sh.at[pl.ds(s * RPT + k * ZR, ZR)])

    plsc.subcore_barrier()

    pltpu.sync_copy(dst_hbm.at[c, s], dst_v)

    def body(j, _):
        pltpu.sync_copy(ones_v, deg_sh.at[dst_v.at[j]], add=True)
        return 0

    lax.fori_loop(0, NCHUNK, body, 0)
    plsc.subcore_barrier()

    @pl.when(s < NWR)
    def _():
        pltpu.sync_copy(deg_sh.at[pl.ds(s * RPT, RPT)],
                        out_hbm.at[c, pl.ds(s * RPT, RPT)])


@functools.partial(
    pl.kernel,
    out_type=jax.ShapeDtypeStruct((NC, N, D), jnp.float32),
    mesh=_MESH,
    scratch_types=[
        pltpu.VMEM((NCHUNK, CH), jnp.int32),
        pltpu.VMEM((2, CH), jnp.int32),
        pltpu.VMEM((2, CH, D), jnp.float32),
        pltpu.VMEM((ZR, D), jnp.float32),
        pltpu.VMEM_SHARED((N, D), jnp.float32),
        pltpu.SemaphoreType.DMA((2,)),
        pltpu.SemaphoreType.DMA((2,)),
    ],
)
def _segsum_kernel(y_hbm, src_hbm, dst_hbm, out_hbm,
                   src_v, dst_v, rows_v, z_v, acc_sh, gsem, dsem):
    c = lax.axis_index("c")
    s = lax.axis_index("s")

    _zero_rows(z_v, ZR, D // 16)

    @pl.when(s < NWR)
    def _():
        for k in range(RPT // ZR):
            pltpu.sync_copy(z_v, acc_sh.at[pl.ds(s * RPT + k * ZR, ZR)])

    plsc.subcore_barrier()

    pltpu.sync_copy(src_hbm.at[c, s], src_v)

    pltpu.async_copy(y_hbm.at[src_v.at[0]], rows_v.at[0], gsem.at[0])
    pltpu.async_copy(dst_hbm.at[c, s, 0], dst_v.at[0], dsem.at[0])

    def body(j, _):
        p = lax.rem(j, 2)
        pn = lax.rem(j + 1, 2)

        @pl.when(j + 1 < NCHUNK)
        def _():
            pltpu.async_copy(y_hbm.at[src_v.at[j + 1]], rows_v.at[pn],
                             gsem.at[pn])
            pltpu.async_copy(dst_hbm.at[c, s, j + 1], dst_v.at[pn],
                             dsem.at[pn])

        pltpu.make_async_copy(y_hbm.at[src_v.at[j]], rows_v.at[p],
                              gsem.at[p]).wait()
        pltpu.make_async_copy(dst_hbm.at[c, s, j], dst_v.at[p],
                              dsem.at[p]).wait()
        pltpu.sync_copy(rows_v.at[p], acc_sh.at[dst_v.at[p]], add=True)
        return 0

    lax.fori_loop(0, NCHUNK, body, 0)
    plsc.subcore_barrier()

    @pl.when(s < NWR)
    def _():
        pltpu.sync_copy(acc_sh.at[pl.ds(s * RPT, RPT)],
                        out_hbm.at[c, pl.ds(s * RPT, RPT)])


RB = 1000


def _dis(d0_ref, d1_ref):
    deg = d0_ref[...][:, 0:1] + d1_ref[...][:, 0:1] + 1.0
    return lax.rsqrt(deg)


def _mm_scale_body(x_ref, w_ref, d0_ref, d1_ref, y_ref):
    dis = _dis(d0_ref, d1_ref)
    y_ref[...] = jnp.dot(x_ref[...], w_ref[...],
                         preferred_element_type=jnp.float32) * dis


def _mid_body(a0_ref, a1_ref, y1_ref, d0_ref, d1_ref, b_ref, w_ref, y2_ref):
    dis = _dis(d0_ref, d1_ref)
    h = dis * (a0_ref[...] + a1_ref[...] + y1_ref[...]) + b_ref[...]
    h = jnp.maximum(h, 0.0)
    y2_ref[...] = jnp.dot(h, w_ref[...],
                          preferred_element_type=jnp.float32) * dis


def _final_body(a0_ref, a1_ref, y2_ref, d0_ref, d1_ref, b_ref, out_ref):
    dis = _dis(d0_ref, d1_ref)
    out_ref[...] = dis * (a0_ref[...] + a1_ref[...] + y2_ref[...]) + b_ref[...]


def _row_spec():
    return pl.BlockSpec((RB, D), lambda i: (i, 0))


def _deg_spec():
    return pl.BlockSpec((RB, 16), lambda i: (i, 0))


def _full_spec(shape):
    return pl.BlockSpec(shape, lambda i: tuple(0 for _ in shape))


def _mm_scale(x, w, d0, d1):
    return pl.pallas_call(
        _mm_scale_body,
        grid=(N // RB,),
        in_specs=[_row_spec(), _full_spec((D, D)), _deg_spec(), _deg_spec()],
        out_specs=_row_spec(),
        out_shape=jax.ShapeDtypeStruct((N, D), jnp.float32),
    )(x, w, d0, d1)


def _mid(a0, a1, y1, d0, d1, b, w):
    return pl.pallas_call(
        _mid_body,
        grid=(N // RB,),
        in_specs=[_row_spec(), _row_spec(), _row_spec(), _deg_spec(),
                  _deg_spec(), _full_spec((1, D)), _full_spec((D, D))],
        out_specs=_row_spec(),
        out_shape=jax.ShapeDtypeStruct((N, D), jnp.float32),
    )(a0, a1, y1, d0, d1, b, w)


def _final(a0, a1, y2, d0, d1, b):
    return pl.pallas_call(
        _final_body,
        grid=(N // RB,),
        in_specs=[_row_spec(), _row_spec(), _row_spec(), _deg_spec(),
                  _deg_spec(), _full_spec((1, D))],
        out_specs=_row_spec(),
        out_shape=jax.ShapeDtypeStruct((N, D), jnp.float32),
    )(a0, a1, y2, d0, d1, b)


def kernel(x, edge_index, W1, b1, W2, b2):
    src = edge_index[0].astype(jnp.int32).reshape(NC, NS, NCHUNK, CH)
    dst = edge_index[1].astype(jnp.int32).reshape(NC, NS, NCHUNK, CH)
    b1r = b1.reshape(1, D)
    b2r = b2.reshape(1, D)

    degp = _deg_kernel(dst)
    d0, d1 = degp[0], degp[1]

    y1 = _mm_scale(x, W1, d0, d1)
    acc1 = _segsum_kernel(y1, src, dst)
    y2 = _mid(acc1[0], acc1[1], y1, d0, d1, b1r, W2)
    acc2 = _segsum_kernel(y2, src, dst)
    return _final(acc2[0], acc2[1], y2, d0, d1, b2r)


# speedup vs baseline: 28.7181x; 1.0806x over previous
import functools

import jax
import jax.numpy as jnp
from jax import lax
from jax.experimental import pallas as pl
from jax.experimental.pallas import tpu as pltpu
from jax.experimental.pallas import tpu_sc as plsc

N = 10000
D = 128
E = 320000
NC = 2
NS = 16
NW = NC * NS
EPW = E // NW
CH = 40
NCHUNK = EPW // CH
NB = 3
DCH = 80
DNCHUNK = EPW // DCH
NWR = 10
RPT = N // NWR
ZR = 8

_MESH = plsc.VectorSubcoreMesh(core_axis_name="c", subcore_axis_name="s")


def _zero_rows(z_ref, nrows, ncol16):
    zv = jnp.zeros((16,), jnp.float32)

    def body(i, _):
        for k in range(ncol16):
            z_ref[i, pl.ds(k * 16, 16)] = zv
        return 0

    lax.fori_loop(0, nrows, body, 0)


@functools.partial(
    pl.kernel,
    out_type=jax.ShapeDtypeStruct((NC, N, 16), jnp.float32),
    mesh=_MESH,
    scratch_types=[
        pltpu.VMEM((DNCHUNK, DCH), jnp.int32),
        pltpu.VMEM((DCH, 16), jnp.float32),
        pltpu.VMEM((ZR, 16), jnp.float32),
        pltpu.VMEM_SHARED((N, 16), jnp.float32),
        pltpu.SemaphoreType.DMA,
    ],
)
def _deg_kernel(dst_hbm, out_hbm, dst_v, ones_v, z_v, deg_sh, sem):
    c = lax.axis_index("c")
    s = lax.axis_index("s")

    ones = jnp.ones((16,), jnp.float32)

    def fill_ones(i, _):
        ones_v[i, pl.ds(0, 16)] = ones
        return 0

    lax.fori_loop(0, DCH, fill_ones, 0)
    _zero_rows(z_v, ZR, 1)

    @pl.when(s < NWR)
    def _():
        for k in range(RPT // ZR):
            pltpu.sync_copy(z_v, deg_sh.at[pl.ds(s * RPT + k * ZR, ZR)])

    plsc.subcore_barrier()

    pltpu.sync_copy(dst_hbm.at[c, s], dst_v)

    W = 8

    def body(j, _):
        pltpu.async_copy(ones_v, deg_sh.at[dst_v.at[j]], sem, add=True)

        @pl.when(j >= W)
        def _():
            pltpu.make_async_copy(ones_v, deg_sh.at[dst_v.at[j]], sem).wait()

        return 0

    lax.fori_loop(0, DNCHUNK, body, 0)

    def drain(j, _):
        pltpu.make_async_copy(ones_v, deg_sh.at[dst_v.at[j]], sem).wait()
        return 0

    lax.fori_loop(0, W, drain, 0)
    plsc.subcore_barrier()

    @pl.when(s < NWR)
    def _():
        pltpu.sync_copy(deg_sh.at[pl.ds(s * RPT, RPT)],
                        out_hbm.at[c, pl.ds(s * RPT, RPT)])


@functools.partial(
    pl.kernel,
    out_type=jax.ShapeDtypeStruct((NC, N, D), jnp.float32),
    mesh=_MESH,
    scratch_types=[
        pltpu.VMEM((NCHUNK, CH), jnp.int32),
        pltpu.VMEM((NB, CH), jnp.int32),
        pltpu.VMEM((NB, CH, D), jnp.float32),
        pltpu.VMEM((ZR, D), jnp.float32),
        pltpu.VMEM_SHARED((N, D), jnp.float32),
        pltpu.SemaphoreType.DMA((NB,)),
        pltpu.SemaphoreType.DMA((NB,)),
        pltpu.SemaphoreType.DMA((NB,)),
    ],
)
def _segsum_kernel(y_hbm, src_hbm, dst_hbm, out_hbm,
                   src_v, dst_v, rows_v, z_v, acc_sh, gsem, dsem, ssem):
    c = lax.axis_index("c")
    s = lax.axis_index("s")

    _zero_rows(z_v, ZR, D // 16)

    @pl.when(s < NWR)
    def _():
        for k in range(RPT // ZR):
            pltpu.sync_copy(z_v, acc_sh.at[pl.ds(s * RPT + k * ZR, ZR)])

    plsc.subcore_barrier()

    pltpu.sync_copy(src_hbm.at[c, s], src_v)

    for b in range(NB - 1):
        pltpu.async_copy(y_hbm.at[src_v.at[b]], rows_v.at[b], gsem.at[b])
        pltpu.async_copy(dst_hbm.at[c, s, b], dst_v.at[b], dsem.at[b])

    def body(j, _):
        p = lax.rem(j, NB)
        pf = lax.rem(j + NB - 1, NB)

        @pl.when(j + NB - 1 < NCHUNK)
        def _():
            @pl.when(j >= 1)
            def _():
                pltpu.make_async_copy(rows_v.at[pf],
                                      acc_sh.at[dst_v.at[pf]],
                                      ssem.at[pf]).wait()

            pltpu.async_copy(y_hbm.at[src_v.at[j + NB - 1]], rows_v.at[pf],
                             gsem.at[pf])
            pltpu.async_copy(dst_hbm.at[c, s, j + NB - 1], dst_v.at[pf],
                             dsem.at[pf])

        pltpu.make_async_copy(y_hbm.at[src_v.at[j]], rows_v.at[p],
                              gsem.at[p]).wait()
        pltpu.make_async_copy(dst_hbm.at[c, s, j], dst_v.at[p],
                              dsem.at[p]).wait()
        pltpu.async_copy(rows_v.at[p], acc_sh.at[dst_v.at[p]], ssem.at[p],
                         add=True)
        return 0

    lax.fori_loop(0, NCHUNK, body, 0)

    def sdrain(b, _):
        pltpu.make_async_copy(rows_v.at[b], acc_sh.at[dst_v.at[b]],
                              ssem.at[b]).wait()
        return 0

    lax.fori_loop(0, NB, sdrain, 0)
    plsc.subcore_barrier()

    @pl.when(s < NWR)
    def _():
        pltpu.sync_copy(acc_sh.at[pl.ds(s * RPT, RPT)],
                        out_hbm.at[c, pl.ds(s * RPT, RPT)])


RB = 1000


def _dis(d0, d1):
    deg = d0[0][:, 0:1] + d1[0][:, 0:1] + 1.0
    return lax.rsqrt(deg)


def _mm_scale_body(x_ref, w_ref, d0_ref, d1_ref, y_ref):
    dis = _dis(d0_ref[...], d1_ref[...])
    y_ref[...] = jnp.dot(x_ref[...], w_ref[...],
                         preferred_element_type=jnp.float32) * dis


def _mid_body(a0_ref, a1_ref, y1_ref, d0_ref, d1_ref, b_ref, w_ref, y2_ref):
    dis = _dis(d0_ref[...], d1_ref[...])
    h = dis * (a0_ref[...][0] + a1_ref[...][0] + y1_ref[...]) + b_ref[...]
    h = jnp.maximum(h, 0.0)
    y2_ref[...] = jnp.dot(h, w_ref[...],
                          preferred_element_type=jnp.float32) * dis


def _final_body(a0_ref, a1_ref, y2_ref, d0_ref, d1_ref, b_ref, out_ref):
    dis = _dis(d0_ref[...], d1_ref[...])
    out_ref[...] = (dis * (a0_ref[...][0] + a1_ref[...][0] + y2_ref[...])
                    + b_ref[...])


def _row_spec():
    return pl.BlockSpec((RB, D), lambda i: (i, 0))


def _acc_spec(core):
    return pl.BlockSpec((1, RB, D), lambda i, c=core: (c, i, 0))


def _deg_spec(core):
    return pl.BlockSpec((1, RB, 16), lambda i, c=core: (c, i, 0))


def _full_spec(shape):
    return pl.BlockSpec(shape, lambda i: tuple(0 for _ in shape))


def _mm_scale(x, w, degp):
    return pl.pallas_call(
        _mm_scale_body,
        grid=(N // RB,),
        in_specs=[_row_spec(), _full_spec((D, D)), _deg_spec(0),
                  _deg_spec(1)],
        out_specs=_row_spec(),
        out_shape=jax.ShapeDtypeStruct((N, D), jnp.float32),
    )(x, w, degp, degp)


def _mid(acc, y1, degp, b, w):
    return pl.pallas_call(
        _mid_body,
        grid=(N // RB,),
        in_specs=[_acc_spec(0), _acc_spec(1), _row_spec(), _deg_spec(0),
                  _deg_spec(1), _full_spec((1, D)), _full_spec((D, D))],
        out_specs=_row_spec(),
        out_shape=jax.ShapeDtypeStruct((N, D), jnp.float32),
    )(acc, acc, y1, degp, degp, b, w)


def _final(acc, y2, degp, b):
    return pl.pallas_call(
        _final_body,
        grid=(N // RB,),
        in_specs=[_acc_spec(0), _acc_spec(1), _row_spec(), _deg_spec(0),
                  _deg_spec(1), _full_spec((1, D))],
        out_specs=_row_spec(),
        out_shape=jax.ShapeDtypeStruct((N, D), jnp.float32),
    )(acc, acc, y2, degp, degp, b)


def kernel(x, edge_index, W1, b1, W2, b2):
    src = edge_index[0].astype(jnp.int32).reshape(NC, NS, NCHUNK, CH)
    dst = edge_index[1].astype(jnp.int32).reshape(NC, NS, NCHUNK, CH)
    dst_d = edge_index[1].astype(jnp.int32).reshape(NC, NS, DNCHUNK, DCH)
    b1r = b1.reshape(1, D)
    b2r = b2.reshape(1, D)

    degp = _deg_kernel(dst_d)
    y1 = _mm_scale(x, W1, degp)
    acc1 = _segsum_kernel(y1, src, dst)
    y2 = _mid(acc1, y1, degp, b1r, W2)
    acc2 = _segsum_kernel(y2, src, dst)
    return _final(acc2, y2, degp, b2r)
